# Initial kernel scaffold; baseline (speedup 1.0000x reference)
import jax
import jax.numpy as jnp
from jax.experimental import pallas as pl


def kernel(x, weight, shared_mat, human_mat, mouse_mat, W_sh, b_sh, W_hu, b_hu, W_mo, b_mo, W_fc, b_fc, W_lin, b_lin, W_d1, b_d1, bn_g, bn_b, W_d2, b_d2, edge_index):
    raise NotImplementedError("write your pallas kernel here")


# speedup vs baseline: 3.1038x; 3.1038x over previous
import functools

import jax
import jax.numpy as jnp
from jax import lax
from jax.experimental import pallas as pl
from jax.experimental.pallas import tpu as pltpu
from jax.experimental.pallas import tpu_sc as plsc

N = 10000
NP = 10240
GENE = 192
NW = 32
CH = 128
NCHUNK = 40
EPT = CH * NCHUNK
E_PAD = NW * EPT
EXW = 8
HALF = 96
STRIPE = NP // 16
LANES = 16


def _sc_feat_body(x1_hbm, x2_hbm, src_hbm, dst_hbm, w_hbm, z96_hbm,
                  out1, out2,
                  acc, src_v, dst_v, w_v, rows_v, sem):
    c = lax.axis_index("c")
    s = lax.axis_index("s")
    wid = s * 2 + c
    base = s * STRIPE

    pltpu.sync_copy(src_hbm.at[wid], src_v)
    pltpu.sync_copy(dst_hbm.at[wid], dst_v)
    pltpu.sync_copy(w_hbm.at[wid], w_v)

    for h, (x_h, out_h) in enumerate(((x1_hbm, out1), (x2_hbm, out2))):
        pltpu.sync_copy(z96_hbm, acc.at[pl.ds(base, STRIPE)])
        plsc.subcore_barrier()

        def chunk(j, _):
            pltpu.async_copy(x_h.at[src_v.at[j]], rows_v, sem).wait()

            def row_group(g, _):
                wv = w_v[j, pl.ds(g * LANES, LANES)]
                for l in range(LANES):
                    wsc = wv[l]
                    i = g * LANES + l
                    for k in range(HALF // LANES):
                        sl = pl.ds(k * LANES, LANES)
                        rows_v[i, sl] = rows_v[i, sl] * wsc
                return 0
            lax.fori_loop(0, CH // LANES, row_group, 0)

            pltpu.sync_copy(rows_v, acc.at[dst_v.at[j]], add=True)
            return 0
        lax.fori_loop(0, NCHUNK, chunk, 0)

        plsc.subcore_barrier()
        pltpu.sync_copy(acc.at[pl.ds(base, STRIPE)], out_h.at[c, pl.ds(base, STRIPE)])
        if h == 0:
            plsc.subcore_barrier()


def _sc_hist_body(ex_hbm, dst_hbm, z8_hbm, out_e, acc2, dst_v, ex_v):
    c = lax.axis_index("c")
    s = lax.axis_index("s")
    wid = s * 2 + c

    base = s * STRIPE
    pltpu.sync_copy(z8_hbm, acc2.at[pl.ds(base, STRIPE)])
    plsc.subcore_barrier()

    pltpu.sync_copy(dst_hbm.at[wid], dst_v)

    def chunk(j, _):
        pltpu.sync_copy(ex_hbm.at[wid, j], ex_v)
        pltpu.sync_copy(ex_v, acc2.at[dst_v.at[j]], add=True)
        return 0
    lax.fori_loop(0, NCHUNK, chunk, 0)

    plsc.subcore_barrier()
    pltpu.sync_copy(acc2.at[pl.ds(base, STRIPE)], out_e.at[c, pl.ds(base, STRIPE)])


def _sc_aggregate(x1, x2, src_r, dst_r, w_r, ex_r, z96, z8):
    mesh = plsc.VectorSubcoreMesh(core_axis_name="c", subcore_axis_name="s")
    feat = pl.kernel(
        _sc_feat_body,
        out_type=[
            jax.ShapeDtypeStruct((2, NP, HALF), jnp.float32),
            jax.ShapeDtypeStruct((2, NP, HALF), jnp.float32),
        ],
        mesh=mesh,
        scratch_types=[
            pltpu.VMEM_SHARED((NP, HALF), jnp.float32),
            pltpu.VMEM((NCHUNK, CH), jnp.int32),
            pltpu.VMEM((NCHUNK, CH), jnp.int32),
            pltpu.VMEM((NCHUNK, CH), jnp.float32),
            pltpu.VMEM((CH, HALF), jnp.float32),
            pltpu.SemaphoreType.DMA,
        ],
        compiler_params=pltpu.CompilerParams(use_tc_tiling_on_sc=False),
        name="gnn_edge_feat_sc",
    )
    hist = pl.kernel(
        _sc_hist_body,
        out_type=jax.ShapeDtypeStruct((2, NP, EXW), jnp.float32),
        mesh=mesh,
        scratch_types=[
            pltpu.VMEM_SHARED((NP, EXW), jnp.float32),
            pltpu.VMEM((NCHUNK, CH), jnp.int32),
            pltpu.VMEM((CH, EXW), jnp.float32),
        ],
        compiler_params=pltpu.CompilerParams(use_tc_tiling_on_sc=False),
        name="gnn_edge_hist_sc",
    )
    s1, s2 = feat(x1, x2, src_r, dst_r, w_r, z96)
    return s1, s2, hist(ex_r, dst_r, z8)


BN = 1000
GRID = N // BN


def _dense_body(x_ref, s1_ref, s2_ref, a_ref, G_ref, g_ref, e_ref, hx_ref, d1_ref, sums_ref):
    x = x_ref[...]
    stot = jnp.concatenate([s1_ref[0] + s1_ref[1], s2_ref[0] + s2_ref[1]], axis=-1)
    a = a_ref[0] + a_ref[1]
    deg = a[:, 0:1]
    wsum = a[:, 1:2]
    inv = 1.0 / (deg + 1.0)
    u = (x + stot) * inv
    cvec = (1.0 + wsum) * inv
    y = jnp.dot(u, G_ref[...], preferred_element_type=jnp.float32)
    y = y + cvec * g_ref[...] + e_ref[...]
    hx_ref[...] = y[:, :16]
    d1 = y[:, 16:66]
    d1_ref[...] = d1
    sums_ref[0, 0, :] = jnp.sum(d1, axis=0)
    sums_ref[0, 1, :] = jnp.sum(d1 * d1, axis=0)


def _bn_body(d1_ref, sums_ref, bng_ref, bnb_ref, W2_ref, b2_ref, out_ref):
    ssum = sums_ref[...]
    mu = jnp.sum(ssum[:, 0, :], axis=0, keepdims=True) / N
    ms = jnp.sum(ssum[:, 1, :], axis=0, keepdims=True) / N
    var = ms - mu * mu
    scale = bng_ref[...] * lax.rsqrt(var + 1e-5)
    d1n = (d1_ref[...] - mu) * scale + bnb_ref[...]
    r = jnp.maximum(d1n, 0.0)
    out_ref[...] = jnp.dot(r, W2_ref[...], preferred_element_type=jnp.float32) + b2_ref[...]


def kernel(x, weight, shared_mat, human_mat, mouse_mat, W_sh, b_sh, W_hu, b_hu,
           W_mo, b_mo, W_fc, b_fc, W_lin, b_lin, W_d1, b_d1, bn_g, bn_b,
           W_d2, b_d2, edge_index):
    f32 = jnp.float32
    src = edge_index[0]
    dst = edge_index[1]
    w = weight[:, 0]
    E = src.shape[0]
    pad = E_PAD - E

    src_r = jnp.concatenate([src, jnp.zeros((pad,), jnp.int32)]).reshape(NW, NCHUNK, CH)
    dst_r = jnp.concatenate([dst, jnp.zeros((pad,), jnp.int32)]).reshape(NW, NCHUNK, CH)
    w_p = jnp.concatenate([w, jnp.zeros((pad,), f32)])
    valid = jnp.concatenate([jnp.ones((E,), f32), jnp.zeros((pad,), f32)])
    ex = jnp.stack([valid, w_p], axis=1)
    ex_r = jnp.pad(ex, ((0, 0), (0, EXW - 2))).reshape(NW, NCHUNK, CH, EXW)
    w_r = w_p.reshape(NW, NCHUNK, CH)
    x1 = x[:, :HALF]
    x2 = x[:, HALF:]
    z96 = jnp.zeros((STRIPE, HALF), f32)
    z8 = jnp.zeros((STRIPE, EXW), f32)

    P = shared_mat @ W_sh.T + human_mat @ W_hu.T + mouse_mat @ W_mo.T
    b_all = b_sh + b_hu + b_mo
    Q = P @ W_fc.T
    v = W_fc @ b_all
    G1 = Q @ W_lin.T
    g1 = W_lin @ v
    e1 = W_lin @ b_fc + b_lin
    G2 = Q @ W_d1.T
    g2 = W_d1 @ v
    e2 = W_d1 @ b_fc + b_d1
    G = jnp.concatenate([G1, G2], axis=1)
    gv = jnp.concatenate([g1, g2])[None, :]
    ev = jnp.concatenate([e1, e2])[None, :]

    s1_parts, s2_parts, e_parts = _sc_aggregate(x1, x2, src_r, dst_r, w_r, ex_r, z96, z8)

    hx, d1, sums = pl.pallas_call(
        _dense_body,
        grid=(GRID,),
        in_specs=[
            pl.BlockSpec((BN, GENE), lambda i: (i, 0)),
            pl.BlockSpec((2, BN, HALF), lambda i: (0, i, 0)),
            pl.BlockSpec((2, BN, HALF), lambda i: (0, i, 0)),
            pl.BlockSpec((2, BN, EXW), lambda i: (0, i, 0)),
            pl.BlockSpec((GENE, 66), lambda i: (0, 0)),
            pl.BlockSpec((1, 66), lambda i: (0, 0)),
            pl.BlockSpec((1, 66), lambda i: (0, 0)),
        ],
        out_specs=[
            pl.BlockSpec((BN, 16), lambda i: (i, 0)),
            pl.BlockSpec((BN, 50), lambda i: (i, 0)),
            pl.BlockSpec((1, 2, 50), lambda i: (i, 0, 0)),
        ],
        out_shape=[
            jax.ShapeDtypeStruct((N, 16), f32),
            jax.ShapeDtypeStruct((N, 50), f32),
            jax.ShapeDtypeStruct((GRID, 2, 50), f32),
        ],
    )(x, s1_parts, s2_parts, e_parts, G, gv, ev)

    domain_x = pl.pallas_call(
        _bn_body,
        grid=(GRID,),
        in_specs=[
            pl.BlockSpec((BN, 50), lambda i: (i, 0)),
            pl.BlockSpec((GRID, 2, 50), lambda i: (0, 0, 0)),
            pl.BlockSpec((1, 50), lambda i: (0, 0)),
            pl.BlockSpec((1, 50), lambda i: (0, 0)),
            pl.BlockSpec((50, 2), lambda i: (0, 0)),
            pl.BlockSpec((1, 2), lambda i: (0, 0)),
        ],
        out_specs=pl.BlockSpec((BN, 2), lambda i: (i, 0)),
        out_shape=jax.ShapeDtypeStruct((N, 2), f32),
    )(d1, sums, bn_g[None, :], bn_b[None, :], W_d2.T, b_d2[None, :])

    return (hx, domain_x)


# speedup vs baseline: 3.2195x; 1.0373x over previous
import functools

import jax
import jax.numpy as jnp
from jax import lax
from jax.experimental import pallas as pl
from jax.experimental.pallas import tpu as pltpu
from jax.experimental.pallas import tpu_sc as plsc

N = 10000
NP = 10240
GENE = 192
NW = 32
CH = 128
NCHUNK = 40
EPT = CH * NCHUNK
E_PAD = NW * EPT
EXW = 8
HALF = 96
STRIPE = NP // 16
LANES = 16


def _sc_feat_body(x1_hbm, x2_hbm, src_hbm, dst_hbm, w_hbm, z96_hbm,
                  out1, out2,
                  acc, src_v, dst_v, w_v, rows0, rows1,
                  gsem0, gsem1, ssem0, ssem1):
    c = lax.axis_index("c")
    s = lax.axis_index("s")
    wid = s * 2 + c
    base = s * STRIPE

    pltpu.sync_copy(src_hbm.at[wid], src_v)
    pltpu.sync_copy(dst_hbm.at[wid], dst_v)
    pltpu.sync_copy(w_hbm.at[wid], w_v)

    def scale(rows, j):
        def row_group(g, _):
            wv = w_v[j, pl.ds(g * LANES, LANES)]
            for l in range(LANES):
                wsc = wv[l]
                i = g * LANES + l
                for k in range(HALF // LANES):
                    sl = pl.ds(k * LANES, LANES)
                    rows[i, sl] = rows[i, sl] * wsc
            return 0
        lax.fori_loop(0, CH // LANES, row_group, 0)

    for h, (x_h, out_h) in enumerate(((x1_hbm, out1), (x2_hbm, out2))):
        pltpu.sync_copy(z96_hbm, acc.at[pl.ds(base, STRIPE)])
        plsc.subcore_barrier()

        pltpu.async_copy(x_h.at[src_v.at[0]], rows0, gsem0)

        def pair(p, _):
            j0 = 2 * p
            j1 = j0 + 1
            pltpu.make_async_copy(x_h.at[src_v.at[j0]], rows0, gsem0).wait()
            scale(rows0, j0)

            @pl.when(p > 0)
            def _():
                pltpu.make_async_copy(rows1, acc.at[dst_v.at[j0 - 1]], ssem1).wait()
            pltpu.async_copy(x_h.at[src_v.at[j1]], rows1, gsem1)
            pltpu.async_copy(rows0, acc.at[dst_v.at[j0]], ssem0, add=True)

            pltpu.make_async_copy(x_h.at[src_v.at[j1]], rows1, gsem1).wait()
            scale(rows1, j1)

            @pl.when(p < NCHUNK // 2 - 1)
            def _():
                pltpu.make_async_copy(rows0, acc.at[dst_v.at[j0]], ssem0).wait()
                pltpu.async_copy(x_h.at[src_v.at[j0 + 2]], rows0, gsem0)
            pltpu.async_copy(rows1, acc.at[dst_v.at[j1]], ssem1, add=True)
            return 0
        lax.fori_loop(0, NCHUNK // 2, pair, 0)

        pltpu.make_async_copy(rows0, acc.at[dst_v.at[NCHUNK - 2]], ssem0).wait()
        pltpu.make_async_copy(rows1, acc.at[dst_v.at[NCHUNK - 1]], ssem1).wait()

        plsc.subcore_barrier()
        pltpu.sync_copy(acc.at[pl.ds(base, STRIPE)], out_h.at[c, pl.ds(base, STRIPE)])
        if h == 0:
            plsc.subcore_barrier()


def _sc_hist_body(ex_hbm, dst_hbm, z8_hbm, out_e, acc2, dst_v, ex_v):
    c = lax.axis_index("c")
    s = lax.axis_index("s")
    wid = s * 2 + c

    base = s * STRIPE
    pltpu.sync_copy(z8_hbm, acc2.at[pl.ds(base, STRIPE)])
    plsc.subcore_barrier()

    pltpu.sync_copy(dst_hbm.at[wid], dst_v)

    def chunk(j, _):
        pltpu.sync_copy(ex_hbm.at[wid, j], ex_v)
        pltpu.sync_copy(ex_v, acc2.at[dst_v.at[j]], add=True)
        return 0
    lax.fori_loop(0, NCHUNK, chunk, 0)

    plsc.subcore_barrier()
    pltpu.sync_copy(acc2.at[pl.ds(base, STRIPE)], out_e.at[c, pl.ds(base, STRIPE)])


def _sc_aggregate(x1, x2, src_r, dst_r, w_r, ex_r, z96, z8):
    mesh = plsc.VectorSubcoreMesh(core_axis_name="c", subcore_axis_name="s")
    feat = pl.kernel(
        _sc_feat_body,
        out_type=[
            jax.ShapeDtypeStruct((2, NP, HALF), jnp.float32),
            jax.ShapeDtypeStruct((2, NP, HALF), jnp.float32),
        ],
        mesh=mesh,
        scratch_types=[
            pltpu.VMEM_SHARED((NP, HALF), jnp.float32),
            pltpu.VMEM((NCHUNK, CH), jnp.int32),
            pltpu.VMEM((NCHUNK, CH), jnp.int32),
            pltpu.VMEM((NCHUNK, CH), jnp.float32),
            pltpu.VMEM((CH, HALF), jnp.float32),
            pltpu.VMEM((CH, HALF), jnp.float32),
            pltpu.SemaphoreType.DMA,
            pltpu.SemaphoreType.DMA,
            pltpu.SemaphoreType.DMA,
            pltpu.SemaphoreType.DMA,
        ],
        compiler_params=pltpu.CompilerParams(use_tc_tiling_on_sc=False),
        name="gnn_edge_feat_sc",
    )
    hist = pl.kernel(
        _sc_hist_body,
        out_type=jax.ShapeDtypeStruct((2, NP, EXW), jnp.float32),
        mesh=mesh,
        scratch_types=[
            pltpu.VMEM_SHARED((NP, EXW), jnp.float32),
            pltpu.VMEM((NCHUNK, CH), jnp.int32),
            pltpu.VMEM((CH, EXW), jnp.float32),
        ],
        compiler_params=pltpu.CompilerParams(use_tc_tiling_on_sc=False),
        name="gnn_edge_hist_sc",
    )
    s1, s2 = feat(x1, x2, src_r, dst_r, w_r, z96)
    return s1, s2, hist(ex_r, dst_r, z8)


BN = 1000
GRID = N // BN


def _dense_body(x_ref, s1_ref, s2_ref, a_ref, G_ref, g_ref, e_ref, hx_ref, d1_ref, sums_ref):
    x = x_ref[...]
    stot = jnp.concatenate([s1_ref[0] + s1_ref[1], s2_ref[0] + s2_ref[1]], axis=-1)
    a = a_ref[0] + a_ref[1]
    deg = a[:, 0:1]
    wsum = a[:, 1:2]
    inv = 1.0 / (deg + 1.0)
    u = (x + stot) * inv
    cvec = (1.0 + wsum) * inv
    y = jnp.dot(u, G_ref[...], preferred_element_type=jnp.float32)
    y = y + cvec * g_ref[...] + e_ref[...]
    hx_ref[...] = y[:, :16]
    d1 = y[:, 16:66]
    d1_ref[...] = d1
    sums_ref[0, 0, :] = jnp.sum(d1, axis=0)
    sums_ref[0, 1, :] = jnp.sum(d1 * d1, axis=0)


def _bn_body(d1_ref, sums_ref, bng_ref, bnb_ref, W2_ref, b2_ref, out_ref):
    ssum = sums_ref[...]
    mu = jnp.sum(ssum[:, 0, :], axis=0, keepdims=True) / N
    ms = jnp.sum(ssum[:, 1, :], axis=0, keepdims=True) / N
    var = ms - mu * mu
    scale = bng_ref[...] * lax.rsqrt(var + 1e-5)
    d1n = (d1_ref[...] - mu) * scale + bnb_ref[...]
    r = jnp.maximum(d1n, 0.0)
    out_ref[...] = jnp.dot(r, W2_ref[...], preferred_element_type=jnp.float32) + b2_ref[...]


def kernel(x, weight, shared_mat, human_mat, mouse_mat, W_sh, b_sh, W_hu, b_hu,
           W_mo, b_mo, W_fc, b_fc, W_lin, b_lin, W_d1, b_d1, bn_g, bn_b,
           W_d2, b_d2, edge_index):
    f32 = jnp.float32
    src = edge_index[0]
    dst = edge_index[1]
    w = weight[:, 0]
    E = src.shape[0]
    pad = E_PAD - E

    src_r = jnp.concatenate([src, jnp.zeros((pad,), jnp.int32)]).reshape(NW, NCHUNK, CH)
    dst_r = jnp.concatenate([dst, jnp.zeros((pad,), jnp.int32)]).reshape(NW, NCHUNK, CH)
    w_p = jnp.concatenate([w, jnp.zeros((pad,), f32)])
    valid = jnp.concatenate([jnp.ones((E,), f32), jnp.zeros((pad,), f32)])
    ex = jnp.stack([valid, w_p], axis=1)
    ex_r = jnp.pad(ex, ((0, 0), (0, EXW - 2))).reshape(NW, NCHUNK, CH, EXW)
    w_r = w_p.reshape(NW, NCHUNK, CH)
    x1 = x[:, :HALF]
    x2 = x[:, HALF:]
    z96 = jnp.zeros((STRIPE, HALF), f32)
    z8 = jnp.zeros((STRIPE, EXW), f32)

    P = shared_mat @ W_sh.T + human_mat @ W_hu.T + mouse_mat @ W_mo.T
    b_all = b_sh + b_hu + b_mo
    Q = P @ W_fc.T
    v = W_fc @ b_all
    G1 = Q @ W_lin.T
    g1 = W_lin @ v
    e1 = W_lin @ b_fc + b_lin
    G2 = Q @ W_d1.T
    g2 = W_d1 @ v
    e2 = W_d1 @ b_fc + b_d1
    G = jnp.concatenate([G1, G2], axis=1)
    gv = jnp.concatenate([g1, g2])[None, :]
    ev = jnp.concatenate([e1, e2])[None, :]

    s1_parts, s2_parts, e_parts = _sc_aggregate(x1, x2, src_r, dst_r, w_r, ex_r, z96, z8)

    hx, d1, sums = pl.pallas_call(
        _dense_body,
        grid=(GRID,),
        in_specs=[
            pl.BlockSpec((BN, GENE), lambda i: (i, 0)),
            pl.BlockSpec((2, BN, HALF), lambda i: (0, i, 0)),
            pl.BlockSpec((2, BN, HALF), lambda i: (0, i, 0)),
            pl.BlockSpec((2, BN, EXW), lambda i: (0, i, 0)),
            pl.BlockSpec((GENE, 66), lambda i: (0, 0)),
            pl.BlockSpec((1, 66), lambda i: (0, 0)),
            pl.BlockSpec((1, 66), lambda i: (0, 0)),
        ],
        out_specs=[
            pl.BlockSpec((BN, 16), lambda i: (i, 0)),
            pl.BlockSpec((BN, 50), lambda i: (i, 0)),
            pl.BlockSpec((1, 2, 50), lambda i: (i, 0, 0)),
        ],
        out_shape=[
            jax.ShapeDtypeStruct((N, 16), f32),
            jax.ShapeDtypeStruct((N, 50), f32),
            jax.ShapeDtypeStruct((GRID, 2, 50), f32),
        ],
    )(x, s1_parts, s2_parts, e_parts, G, gv, ev)

    domain_x = pl.pallas_call(
        _bn_body,
        grid=(GRID,),
        in_specs=[
            pl.BlockSpec((BN, 50), lambda i: (i, 0)),
            pl.BlockSpec((GRID, 2, 50), lambda i: (0, 0, 0)),
            pl.BlockSpec((1, 50), lambda i: (0, 0)),
            pl.BlockSpec((1, 50), lambda i: (0, 0)),
            pl.BlockSpec((50, 2), lambda i: (0, 0)),
            pl.BlockSpec((1, 2), lambda i: (0, 0)),
        ],
        out_specs=pl.BlockSpec((BN, 2), lambda i: (i, 0)),
        out_shape=jax.ShapeDtypeStruct((N, 2), f32),
    )(d1, sums, bn_g[None, :], bn_b[None, :], W_d2.T, b_d2[None, :])

    return (hx, domain_x)


# speedup vs baseline: 5.9118x; 1.8362x over previous
import functools

import jax
import jax.numpy as jnp
from jax import lax
from jax.experimental import pallas as pl
from jax.experimental.pallas import tpu as pltpu
from jax.experimental.pallas import tpu_sc as plsc

N = 10000
NP = 10240
GENE = 192
NW = 32
CH = 128
NCHUNK = 40
EPT = CH * NCHUNK
E_PAD = NW * EPT
EXW = 8
OUTC = 66
WIDE = 96
STRIPE = NP // 16
LANES = 16


def _sc_feat_body(xg_hbm, src_hbm, dst_hbm, w_hbm, zw_hbm, out_s,
                  acc, src_v, dst_v, w_v, rows0, rows1, rows2, rows3,
                  gsem0, gsem1, gsem2, gsem3, ssem0, ssem1, ssem2, ssem3):
    c = lax.axis_index("c")
    s = lax.axis_index("s")
    wid = s * 2 + c
    base = s * STRIPE

    pltpu.sync_copy(src_hbm.at[wid], src_v)
    pltpu.sync_copy(dst_hbm.at[wid], dst_v)
    pltpu.sync_copy(w_hbm.at[wid], w_v)

    def scale(rows, j):
        def row_group(g, _):
            wv = w_v[j, pl.ds(g * LANES, LANES)]
            for l in range(LANES):
                ws16 = jnp.full((LANES,), wv[l], dtype=jnp.float32)
                wsb = plsc.pack(ws16, ws16, format=plsc.PackFormat.INTERLEAVED)
                i = g * LANES + l
                for k in range(WIDE // (2 * LANES)):
                    sl = pl.ds(k * 2 * LANES, 2 * LANES)
                    rows[i, sl] = rows[i, sl] * wsb
            return 0
        lax.fori_loop(0, CH // LANES, row_group, 0)

    pltpu.sync_copy(zw_hbm, acc.at[pl.ds(base, STRIPE)])
    plsc.subcore_barrier()

    rows = (rows0, rows1, rows2, rows3)
    gsem = (gsem0, gsem1, gsem2, gsem3)
    ssem = (ssem0, ssem1, ssem2, ssem3)
    pltpu.async_copy(xg_hbm.at[src_v.at[0]], rows0, gsem0)

    def quad(p, _):
        for b in range(4):
            j = 4 * p + b
            bn = (b + 1) % 4
            pltpu.make_async_copy(xg_hbm.at[src_v.at[j]], rows[b], gsem[b]).wait()

            @pl.when(j >= 3)
            def _():
                pltpu.make_async_copy(rows[bn], acc.at[dst_v.at[j - 3]], ssem[bn]).wait()

            @pl.when(j + 1 < NCHUNK)
            def _():
                pltpu.async_copy(xg_hbm.at[src_v.at[j + 1]], rows[bn], gsem[bn])
            scale(rows[b], j)
            pltpu.async_copy(rows[b], acc.at[dst_v.at[j]], ssem[b], add=True)
        return 0
    lax.fori_loop(0, NCHUNK // 4, quad, 0)

    for j in (NCHUNK - 3, NCHUNK - 2, NCHUNK - 1):
        b = j % 4
        pltpu.make_async_copy(rows[b], acc.at[dst_v.at[j]], ssem[b]).wait()

    plsc.subcore_barrier()
    pltpu.sync_copy(acc.at[pl.ds(base, STRIPE)], out_s.at[c, pl.ds(base, STRIPE)])


def _sc_hist_body(ex_hbm, dst_hbm, z8_hbm, out_e, acc2, dst_v, ex_v):
    c = lax.axis_index("c")
    s = lax.axis_index("s")
    wid = s * 2 + c

    base = s * STRIPE
    pltpu.sync_copy(z8_hbm, acc2.at[pl.ds(base, STRIPE)])
    plsc.subcore_barrier()

    pltpu.sync_copy(dst_hbm.at[wid], dst_v)

    def chunk(j, _):
        pltpu.sync_copy(ex_hbm.at[wid, j], ex_v)
        pltpu.sync_copy(ex_v, acc2.at[dst_v.at[j]], add=True)
        return 0
    lax.fori_loop(0, NCHUNK, chunk, 0)

    plsc.subcore_barrier()
    pltpu.sync_copy(acc2.at[pl.ds(base, STRIPE)], out_e.at[c, pl.ds(base, STRIPE)])


def _sc_aggregate(xg, src_r, dst_r, w_r, ex_r, zw, z8):
    mesh = plsc.VectorSubcoreMesh(core_axis_name="c", subcore_axis_name="s")
    feat = pl.kernel(
        _sc_feat_body,
        out_type=jax.ShapeDtypeStruct((2, NP, WIDE), jnp.bfloat16),
        mesh=mesh,
        scratch_types=[
            pltpu.VMEM_SHARED((NP, WIDE), jnp.bfloat16),
            pltpu.VMEM((NCHUNK, CH), jnp.int32),
            pltpu.VMEM((NCHUNK, CH), jnp.int32),
            pltpu.VMEM((NCHUNK, CH), jnp.float32),
            pltpu.VMEM((CH, WIDE), jnp.bfloat16),
            pltpu.VMEM((CH, WIDE), jnp.bfloat16),
            pltpu.VMEM((CH, WIDE), jnp.bfloat16),
            pltpu.VMEM((CH, WIDE), jnp.bfloat16),
            pltpu.SemaphoreType.DMA,
            pltpu.SemaphoreType.DMA,
            pltpu.SemaphoreType.DMA,
            pltpu.SemaphoreType.DMA,
            pltpu.SemaphoreType.DMA,
            pltpu.SemaphoreType.DMA,
            pltpu.SemaphoreType.DMA,
            pltpu.SemaphoreType.DMA,
        ],
        compiler_params=pltpu.CompilerParams(
            use_tc_tiling_on_sc=False, needs_layout_passes=False),
        name="gnn_edge_feat_sc",
    )
    hist = pl.kernel(
        _sc_hist_body,
        out_type=jax.ShapeDtypeStruct((2, NP, EXW), jnp.float32),
        mesh=mesh,
        scratch_types=[
            pltpu.VMEM_SHARED((NP, EXW), jnp.float32),
            pltpu.VMEM((NCHUNK, CH), jnp.int32),
            pltpu.VMEM((CH, EXW), jnp.float32),
        ],
        compiler_params=pltpu.CompilerParams(use_tc_tiling_on_sc=False),
        name="gnn_edge_hist_sc",
    )
    return feat(xg, src_r, dst_r, w_r, zw), hist(ex_r, dst_r, z8)


BN = 1000
GRID = N // BN


def _proj_body(x_ref, G_ref, xg_ref, xgb_ref):
    y = jnp.dot(x_ref[...], G_ref[...], preferred_element_type=jnp.float32)
    xg_ref[...] = y
    xgb_ref[...] = y.astype(jnp.bfloat16)


def _dense_body(xg_ref, s_ref, a_ref, g_ref, e_ref, hx_ref, d1_ref, sums_ref):
    stot = s_ref[0].astype(jnp.float32) + s_ref[1].astype(jnp.float32)
    a = a_ref[0] + a_ref[1]
    deg = a[:, 0:1]
    wsum = a[:, 1:2]
    inv = 1.0 / (deg + 1.0)
    cvec = (1.0 + wsum) * inv
    y = (xg_ref[...] + stot) * inv + cvec * g_ref[...] + e_ref[...]
    hx_ref[...] = y[:, :16]
    d1 = y[:, 16:66]
    d1_ref[...] = d1
    sums_ref[0, 0, :] = jnp.sum(d1, axis=0)
    sums_ref[0, 1, :] = jnp.sum(d1 * d1, axis=0)


def _bn_body(d1_ref, sums_ref, bng_ref, bnb_ref, W2_ref, b2_ref, out_ref):
    ssum = sums_ref[...]
    mu = jnp.sum(ssum[:, 0, :], axis=0, keepdims=True) / N
    ms = jnp.sum(ssum[:, 1, :], axis=0, keepdims=True) / N
    var = ms - mu * mu
    scale = bng_ref[...] * lax.rsqrt(var + 1e-5)
    d1n = (d1_ref[...] - mu) * scale + bnb_ref[...]
    r = jnp.maximum(d1n, 0.0)
    out_ref[...] = jnp.dot(r, W2_ref[...], preferred_element_type=jnp.float32) + b2_ref[...]


def kernel(x, weight, shared_mat, human_mat, mouse_mat, W_sh, b_sh, W_hu, b_hu,
           W_mo, b_mo, W_fc, b_fc, W_lin, b_lin, W_d1, b_d1, bn_g, bn_b,
           W_d2, b_d2, edge_index):
    f32 = jnp.float32
    src = edge_index[0]
    dst = edge_index[1]
    w = weight[:, 0]
    E = src.shape[0]
    pad = E_PAD - E

    src_r = jnp.concatenate([src, jnp.zeros((pad,), jnp.int32)]).reshape(NW, NCHUNK, CH)
    dst_r = jnp.concatenate([dst, jnp.zeros((pad,), jnp.int32)]).reshape(NW, NCHUNK, CH)
    w_p = jnp.concatenate([w, jnp.zeros((pad,), f32)])
    valid = jnp.concatenate([jnp.ones((E,), f32), jnp.zeros((pad,), f32)])
    ex = jnp.stack([valid, w_p], axis=1)
    ex_r = jnp.pad(ex, ((0, 0), (0, EXW - 2))).reshape(NW, NCHUNK, CH, EXW)
    w_r = w_p.reshape(NW, NCHUNK, CH)
    zw = jnp.zeros((STRIPE, WIDE), jnp.bfloat16)
    z8 = jnp.zeros((STRIPE, EXW), f32)

    P = shared_mat @ W_sh.T + human_mat @ W_hu.T + mouse_mat @ W_mo.T
    b_all = b_sh + b_hu + b_mo
    Q = P @ W_fc.T
    v = W_fc @ b_all
    G1 = Q @ W_lin.T
    g1 = W_lin @ v
    e1 = W_lin @ b_fc + b_lin
    G2 = Q @ W_d1.T
    g2 = W_d1 @ v
    e2 = W_d1 @ b_fc + b_d1
    G = jnp.concatenate([G1, G2], axis=1)
    Gp = jnp.pad(G, ((0, 0), (0, WIDE - OUTC)))
    gv = jnp.pad(jnp.concatenate([g1, g2]), (0, WIDE - OUTC))[None, :]
    ev = jnp.pad(jnp.concatenate([e1, e2]), (0, WIDE - OUTC))[None, :]

    xg, xgb = pl.pallas_call(
        _proj_body,
        grid=(GRID,),
        in_specs=[
            pl.BlockSpec((BN, GENE), lambda i: (i, 0)),
            pl.BlockSpec((GENE, WIDE), lambda i: (0, 0)),
        ],
        out_specs=[
            pl.BlockSpec((BN, WIDE), lambda i: (i, 0)),
            pl.BlockSpec((BN, WIDE), lambda i: (i, 0)),
        ],
        out_shape=[
            jax.ShapeDtypeStruct((N, WIDE), f32),
            jax.ShapeDtypeStruct((N, WIDE), jnp.bfloat16),
        ],
    )(x, Gp)

    s_parts, e_parts = _sc_aggregate(xgb, src_r, dst_r, w_r, ex_r, zw, z8)

    hx, d1, sums = pl.pallas_call(
        _dense_body,
        grid=(GRID,),
        in_specs=[
            pl.BlockSpec((BN, WIDE), lambda i: (i, 0)),
            pl.BlockSpec((2, BN, WIDE), lambda i: (0, i, 0)),
            pl.BlockSpec((2, BN, EXW), lambda i: (0, i, 0)),
            pl.BlockSpec((1, WIDE), lambda i: (0, 0)),
            pl.BlockSpec((1, WIDE), lambda i: (0, 0)),
        ],
        out_specs=[
            pl.BlockSpec((BN, 16), lambda i: (i, 0)),
            pl.BlockSpec((BN, 50), lambda i: (i, 0)),
            pl.BlockSpec((1, 2, 50), lambda i: (i, 0, 0)),
        ],
        out_shape=[
            jax.ShapeDtypeStruct((N, 16), f32),
            jax.ShapeDtypeStruct((N, 50), f32),
            jax.ShapeDtypeStruct((GRID, 2, 50), f32),
        ],
    )(xg, s_parts, e_parts, gv, ev)

    domain_x = pl.pallas_call(
        _bn_body,
        grid=(GRID,),
        in_specs=[
            pl.BlockSpec((BN, 50), lambda i: (i, 0)),
            pl.BlockSpec((GRID, 2, 50), lambda i: (0, 0, 0)),
            pl.BlockSpec((1, 50), lambda i: (0, 0)),
            pl.BlockSpec((1, 50), lambda i: (0, 0)),
            pl.BlockSpec((50, 2), lambda i: (0, 0)),
            pl.BlockSpec((1, 2), lambda i: (0, 0)),
        ],
        out_specs=pl.BlockSpec((BN, 2), lambda i: (i, 0)),
        out_shape=jax.ShapeDtypeStruct((N, 2), f32),
    )(d1, sums, bn_g[None, :], bn_b[None, :], W_d2.T, b_d2[None, :])

    return (hx, domain_x)
